# baseline (speedup 1.0000x reference)
import functools

import jax
import jax.numpy as jnp
from jax import lax
from jax.experimental import pallas as pl
from jax.experimental.pallas import tpu as pltpu
from jax.experimental.pallas import tpu_sc as plsc

VOCAB = 100000
D = 320
K = 20
NUM_CHUNKS = 50
B = 128
R = NUM_CHUNKS * B
NW = 32
ROWS_PER_W = R // NW
G = 4
NB = ROWS_PER_W // G
LANES = 16
NV = D // LANES


def _sc_kernel(idx_hbm, table_hbm, out_hbm, idx_v, rows_v, acc_v, sem):
    c = lax.axis_index("c")
    s = lax.axis_index("s")
    wid = s * 2 + c
    base = wid * ROWS_PER_W

    def body(b, carry):
        r0 = base + b * G
        pltpu.sync_copy(idx_hbm.at[pl.ds(r0 * K, G * K)], idx_v)
        pltpu.async_copy(table_hbm.at[idx_v], rows_v, sem).wait()
        for g in range(G):
            for v in range(NV):
                col = pl.ds(v * LANES, LANES)
                a = rows_v[g * K, col]
                for p in range(1, K):
                    a = a + rows_v[g * K + p, col]
                acc_v[g, col] = a * (1.0 / K)
        pltpu.sync_copy(acc_v, out_hbm.at[pl.ds(r0, G)])
        return carry

    lax.fori_loop(0, NB, body, 0)


@jax.jit
def kernel(tokens, table):
    flat_idx = (
        tokens.astype(jnp.int32)
        .reshape(NUM_CHUNKS, K, B)
        .transpose(0, 2, 1)
        .reshape(R * K)
    )
    mesh = plsc.VectorSubcoreMesh(core_axis_name="c", subcore_axis_name="s")
    out = pl.kernel(
        _sc_kernel,
        out_type=jax.ShapeDtypeStruct((R, D), jnp.float32),
        mesh=mesh,
        scratch_types=[
            pltpu.VMEM((G * K,), jnp.int32),
            pltpu.VMEM((G * K, D), jnp.float32),
            pltpu.VMEM((G, D), jnp.float32),
            pltpu.SemaphoreType.DMA,
        ],
        compiler_params=pltpu.CompilerParams(use_tc_tiling_on_sc=False),
    )(flat_idx, table)
    return out.reshape(NUM_CHUNKS, B, D)
